# baseline (speedup 1.0000x reference)
import functools

import jax
import jax.numpy as jnp
from jax import lax
from jax.experimental import pallas as pl
from jax.experimental.pallas import tpu as pltpu
from jax.experimental.pallas import tpu_sc as plsc

N = 10000
NPAD = 10240
D_IN = 128
HID = 64
OUT = 64
NHEAD = 8
DH = OUT // NHEAD
FF = 2048
NMP = 3

CHUNK = 128
NBUF = 4
ZROWS = 64
ROWS_PER_TILE = NPAD // 16

EDGES_PER_TILE = 20480
EDGES_PADDED = EDGES_PER_TILE * 16
REL_ROWS = EDGES_PADDED // CHUNK
TILE_ROWS = EDGES_PER_TILE // CHUNK
NFILLS = TILE_ROWS


def _sc_scatter_body(xtab, srcs, dsts, out, src_v, dst_v,
                     buf0, buf1, buf2, buf3,
                     zeros_v, acc, sg0, sg1, sg2, sg3, ss0, ss1, ss2, ss3):
    cid = lax.axis_index("c")
    sid = lax.axis_index("s")

    def _zrow(r, carry):
        for c4 in range(HID // 16):
            zeros_v[r, pl.ds(c4 * 16, 16)] = jnp.zeros((16,), jnp.float32)
        return carry

    lax.fori_loop(0, ZROWS, _zrow, 0)

    bufs = (buf0, buf1, buf2, buf3)
    gsems = (sg0, sg1, sg2, sg3)
    ssems = (ss0, ss1, ss2, ss3)

    def _fire_gather(f, p):
        pltpu.async_copy(xtab.at[src_v.at[f]], bufs[p], gsems[p])

    def _wait_gather(f, p):
        pltpu.make_async_copy(xtab.at[src_v.at[f]], bufs[p], gsems[p]).wait()

    def _fire_scatter(f, p):
        pltpu.async_copy(bufs[p], acc.at[dst_v.at[f]], ssems[p], add=True)

    def _wait_scatter(f, p):
        pltpu.make_async_copy(bufs[p], acc.at[dst_v.at[f]], ssems[p]).wait()

    for m in range(NMP):
        def _zacc(j, carry):
            pltpu.sync_copy(
                zeros_v,
                acc.at[pl.ds(sid * ROWS_PER_TILE + j * ZROWS, ZROWS)])
            return carry

        lax.fori_loop(0, ROWS_PER_TILE // ZROWS, _zacc, 0)

        seg = m * REL_ROWS + sid * TILE_ROWS
        pltpu.sync_copy(srcs.at[cid, pl.ds(seg, TILE_ROWS)], src_v)
        pltpu.sync_copy(dsts.at[pl.ds(seg, TILE_ROWS)], dst_v)
        plsc.subcore_barrier()

        for p in range(NBUF - 1):
            _fire_gather(p, p)

        @pl.loop(0, NFILLS, step=NBUF)
        def _round(f0):
            for p in range(NBUF):
                f = f0 + p
                _wait_gather(f, p)
                _fire_scatter(f, p)

                @pl.when(f >= 1)
                def _():
                    _wait_scatter(f - 1, (p - 1) % NBUF)

                @pl.when(f + NBUF - 1 < NFILLS)
                def _():
                    _fire_gather(f + NBUF - 1, (p + NBUF - 1) % NBUF)

        _wait_scatter(NFILLS - 1, (NFILLS - 1) % NBUF)
        plsc.subcore_barrier()

        pltpu.sync_copy(
            acc.at[pl.ds(sid * ROWS_PER_TILE, ROWS_PER_TILE)],
            out.at[cid, m, pl.ds(sid * ROWS_PER_TILE, ROWS_PER_TILE)],
        )


_SC_SCATTER_CACHE = []


def _sc_scatter(xtab, srcs, dsts):
    if not _SC_SCATTER_CACHE:
        fn = functools.partial(
            pl.kernel,
            out_type=jax.ShapeDtypeStruct((2, NMP, NPAD, HID), jnp.float32),
            mesh=plsc.VectorSubcoreMesh(
                core_axis_name="c", subcore_axis_name="s"),
            compiler_params=pltpu.CompilerParams(use_tc_tiling_on_sc=False),
            scratch_types=[
                pltpu.VMEM((TILE_ROWS, CHUNK), jnp.int32),
                pltpu.VMEM((TILE_ROWS, CHUNK), jnp.int32),
            ] + [pltpu.VMEM((CHUNK, HID), jnp.float32) for _ in range(NBUF)] + [
                pltpu.VMEM((ZROWS, HID), jnp.float32),
                pltpu.VMEM_SHARED((NPAD, HID), jnp.float32),
            ] + [pltpu.SemaphoreType.DMA for _ in range(2 * NBUF)],
        )(_sc_scatter_body)
        _SC_SCATTER_CACHE.append(fn)
    return _SC_SCATTER_CACHE[0](xtab, srcs, dsts)


_B2 = 1000


def _ln(v, s, b):
    mu = jnp.mean(v, axis=-1, keepdims=True)
    var = jnp.mean((v - mu) ** 2, axis=-1, keepdims=True)
    return (v - mu) / jnp.sqrt(var + 1e-5) * s + b


def _bf(a):
    return a.astype(jnp.bfloat16)


def _bdot(a, b):
    return jnp.dot(_bf(a), _bf(b), preferred_element_type=jnp.float32)


def _post_body(x_ref, agg_ref, wl_ref, w0_ref, w1_ref, b0_ref,
               outw_ref, outb_ref,
               wq_ref, bq_ref, lns_ref, lnb_ref,
               ff1_ref, ff1b_ref, ff2_ref, ff2b_ref,
               h1_ref, h1b_ref, h2_ref, h2b_ref, o_ref):
    x = x_ref[...]
    zs = []
    for m in range(NMP):
        aggm = jnp.concatenate([agg_ref[0, m], agg_ref[1, m]], axis=-1)
        hm = _bdot(aggm, wl_ref[m]) + _bdot(x, w0_ref[m]) + _bdot(x, w1_ref[m])
        hm = jnp.maximum(hm + b0_ref[m], 0.0)
        em = _bdot(hm, outw_ref[m])
        zs.append((em + outb_ref[m]) * (1.0 / NMP))

    gi = lax.broadcasted_iota(jnp.int32, (OUT, NHEAD), 0) // DH
    gh = lax.broadcasted_iota(jnp.int32, (OUT, NHEAD), 1)
    G = (gi == gh).astype(jnp.float32)
    sqrt_dh = jnp.float32(DH ** 0.5)

    for l in range(2):
        q = [_bdot(z, wq_ref[l, 0]) + bq_ref[l, 0] for z in zs]
        k = [_bdot(z, wq_ref[l, 1]) + bq_ref[l, 1] for z in zs]
        v = [_bdot(z, wq_ref[l, 2]) + bq_ref[l, 2] for z in zs]
        qb = [_bf(a).astype(jnp.float32) for a in q]
        kb = [_bf(a).astype(jnp.float32) for a in k]
        vb = [_bf(a).astype(jnp.float32) for a in v]
        new_zs = []
        for i in range(NMP):
            s_ij = [jnp.dot(qb[i] * kb[j], G, preferred_element_type=jnp.float32) / sqrt_dh
                    for j in range(NMP)]
            smax = jnp.maximum(jnp.maximum(s_ij[0], s_ij[1]), s_ij[2])
            e_ij = [jnp.exp(s - smax) for s in s_ij]
            den = e_ij[0] + e_ij[1] + e_ij[2]
            o_acc = jnp.zeros_like(zs[i])
            for j in range(NMP):
                w = jnp.dot(_bf(e_ij[j] / den).astype(jnp.float32), G.T,
                            preferred_element_type=jnp.float32)
                o_acc = o_acc + w * vb[j]
            attn = _bdot(o_acc, wq_ref[l, 3]) + bq_ref[l, 3]
            new_zs.append(_ln(zs[i] + attn, lns_ref[l, 0], lnb_ref[l, 0]))
        zcat = jnp.concatenate(new_zs, axis=0)
        f = jnp.maximum(_bdot(zcat, ff1_ref[l]) + ff1b_ref[l], 0.0)
        f = _bdot(f, ff2_ref[l]) + ff2b_ref[l]
        zs = [_ln(new_zs[i] + f[i * _B2:(i + 1) * _B2], lns_ref[l, 1], lnb_ref[l, 1])
              for i in range(NMP)]

    pooled = (zs[0] + zs[1] + zs[2]) / jnp.float32(NMP)
    h = jnp.maximum(_bdot(pooled, h1_ref[...]) + h1b_ref[...], 0.0)
    o_ref[...] = _bdot(h, h2_ref[...]) + h2b_ref[...]


def _full(shape):
    nd = len(shape)
    return pl.BlockSpec(shape, lambda i, _n=nd: (0,) * _n)


def _post(x, agg, weights):
    in_specs = [
        pl.BlockSpec((_B2, D_IN), lambda i: (i, 0)),
        pl.BlockSpec((2, NMP, _B2, HID), lambda i: (0, 0, i, 0)),
    ] + [_full(w.shape) for w in weights]
    return pl.pallas_call(
        _post_body,
        grid=(N // _B2,),
        in_specs=in_specs,
        out_specs=pl.BlockSpec((_B2, 1), lambda i: (i, 0)),
        out_shape=jax.ShapeDtypeStruct((N, 1), jnp.float32),
    )(x, agg, *weights)


def _stack_weights(params):
    mps = [params['mp%d' % m] for m in range(NMP)]
    wl = jnp.stack([p['wl_W'] for p in mps])
    w0 = jnp.stack([p['w0_W'] for p in mps])
    w1 = jnp.stack([p['w1_W'] for p in mps])
    b0 = jnp.stack([(p['w0_b'] + p['w1_b'] + p['wl_b'])[None, :] for p in mps])
    outw = jnp.stack([p['out_W'] for p in mps])
    outb = jnp.stack([p['out_b'][None, :] for p in mps])
    encs = [params['enc%d' % l] for l in range(2)]
    wq = jnp.stack([jnp.stack([p['W' + nm] for nm in ('q', 'k', 'v', 'o')]) for p in encs])
    bq = jnp.stack([jnp.stack([p['b' + nm][None, :] for nm in ('q', 'k', 'v', 'o')]) for p in encs])
    lns = jnp.stack([jnp.stack([p['ln1_s'][None, :], p['ln2_s'][None, :]]) for p in encs])
    lnb = jnp.stack([jnp.stack([p['ln1_b'][None, :], p['ln2_b'][None, :]]) for p in encs])
    ff1 = jnp.stack([p['ff1_W'] for p in encs])
    ff1b = jnp.stack([p['ff1_b'][None, :] for p in encs])
    ff2 = jnp.stack([p['ff2_W'] for p in encs])
    ff2b = jnp.stack([p['ff2_b'][None, :] for p in encs])
    h1 = params['head1_W']
    h1b = params['head1_b'][None, :]
    h2 = params['head2_W']
    h2b = params['head2_b'][None, :]
    return (wl, w0, w1, b0, outw, outb, wq, bq, lns, lnb,
            ff1, ff1b, ff2, ff2b, h1, h1b, h2, h2b)


def _edge_lists(e0, e1, e2):
    es = [e0, e1, e2]
    pad = EDGES_PADDED - es[0].shape[1]
    base = jnp.concatenate(
        [jnp.concatenate([es[m][1].astype(jnp.int32),
                          jnp.zeros((pad,), jnp.int32)])
         for m in range(NMP)])
    srcs = jnp.stack([2 * base, 2 * base + 1])
    sink = N + jnp.arange(pad, dtype=jnp.int32) % (NPAD - N)
    dsts = jnp.concatenate(
        [jnp.concatenate([es[m][0].astype(jnp.int32), sink])
         for m in range(NMP)])
    return srcs.reshape(2, -1, CHUNK), dsts.reshape(-1, CHUNK)


def kernel(x, edge_index_rel0, edge_index_rel1, edge_index_rel2, params):
    xtab = x.reshape(2 * N, HID)
    srcs, dsts = _edge_lists(edge_index_rel0, edge_index_rel1, edge_index_rel2)
    agg = _sc_scatter(xtab, srcs, dsts)
    out = _post(x, agg, _stack_weights(params))
    return out[:, 0]


# speedup vs baseline: 1.0012x; 1.0012x over previous
import functools

import jax
import jax.numpy as jnp
from jax import lax
from jax.experimental import pallas as pl
from jax.experimental.pallas import tpu as pltpu
from jax.experimental.pallas import tpu_sc as plsc

N = 10000
NPAD = 10240
D_IN = 128
HID = 64
OUT = 64
NHEAD = 8
DH = OUT // NHEAD
FF = 2048
NMP = 3

CHUNK = 128
NBUF = 4
ZROWS = 64
ROWS_PER_TILE = NPAD // 16

EDGES_PER_TILE = 20480
EDGES_PADDED = EDGES_PER_TILE * 16
REL_ROWS = EDGES_PADDED // CHUNK
TILE_ROWS = EDGES_PER_TILE // CHUNK
NFILLS = TILE_ROWS


def _sc_scatter_body(xtab, srcs, dsts, out, src_v, dst_v,
                     buf0, buf1, buf2, buf3,
                     zeros_v, acc, sg0, sg1, sg2, sg3, ss0, ss1, ss2, ss3):
    cid = lax.axis_index("c")
    sid = lax.axis_index("s")

    def _zrow(r, carry):
        for c4 in range(HID // 16):
            zeros_v[r, pl.ds(c4 * 16, 16)] = jnp.zeros((16,), jnp.float32)
        return carry

    lax.fori_loop(0, ZROWS, _zrow, 0)

    bufs = (buf0, buf1, buf2, buf3)
    gsems = (sg0, sg1, sg2, sg3)
    ssems = (ss0, ss1, ss2, ss3)

    def _fire_gather(f, p):
        pltpu.async_copy(xtab.at[src_v.at[f]], bufs[p], gsems[p])

    def _wait_gather(f, p):
        pltpu.make_async_copy(xtab.at[src_v.at[f]], bufs[p], gsems[p]).wait()

    def _fire_scatter(f, p):
        pltpu.async_copy(bufs[p], acc.at[dst_v.at[f]], ssems[p], add=True)

    def _wait_scatter(f, p):
        pltpu.make_async_copy(bufs[p], acc.at[dst_v.at[f]], ssems[p]).wait()

    for m in range(NMP):
        def _zacc(j, carry):
            pltpu.sync_copy(
                zeros_v,
                acc.at[pl.ds(sid * ROWS_PER_TILE + j * ZROWS, ZROWS)])
            return carry

        lax.fori_loop(0, ROWS_PER_TILE // ZROWS, _zacc, 0)

        seg = m * REL_ROWS + sid * TILE_ROWS
        pltpu.sync_copy(srcs.at[cid, pl.ds(seg, TILE_ROWS)], src_v)
        pltpu.sync_copy(dsts.at[pl.ds(seg, TILE_ROWS)], dst_v)
        plsc.subcore_barrier()

        for p in range(NBUF - 1):
            _fire_gather(p, p)

        @pl.loop(0, NFILLS, step=NBUF)
        def _round(f0):
            for p in range(NBUF):
                f = f0 + p
                _wait_gather(f, p)
                _fire_scatter(f, p)

                @pl.when(f >= 1)
                def _():
                    _wait_scatter(f - 1, (p - 1) % NBUF)

                @pl.when(f + NBUF - 1 < NFILLS)
                def _():
                    _fire_gather(f + NBUF - 1, (p + NBUF - 1) % NBUF)

        _wait_scatter(NFILLS - 1, (NFILLS - 1) % NBUF)
        plsc.subcore_barrier()

        pltpu.sync_copy(
            acc.at[pl.ds(sid * ROWS_PER_TILE, ROWS_PER_TILE)],
            out.at[cid, m, pl.ds(sid * ROWS_PER_TILE, ROWS_PER_TILE)],
        )


_SC_SCATTER_CACHE = []


def _sc_scatter(xtab, srcs, dsts):
    if not _SC_SCATTER_CACHE:
        fn = functools.partial(
            pl.kernel,
            out_type=jax.ShapeDtypeStruct((2, NMP, NPAD, HID), jnp.float32),
            mesh=plsc.VectorSubcoreMesh(
                core_axis_name="c", subcore_axis_name="s"),
            compiler_params=pltpu.CompilerParams(use_tc_tiling_on_sc=False),
            scratch_types=[
                pltpu.VMEM((TILE_ROWS, CHUNK), jnp.int32),
                pltpu.VMEM((TILE_ROWS, CHUNK), jnp.int32),
            ] + [pltpu.VMEM((CHUNK, HID), jnp.float32) for _ in range(NBUF)] + [
                pltpu.VMEM((ZROWS, HID), jnp.float32),
                pltpu.VMEM_SHARED((NPAD, HID), jnp.float32),
            ] + [pltpu.SemaphoreType.DMA for _ in range(2 * NBUF)],
        )(_sc_scatter_body)
        _SC_SCATTER_CACHE.append(fn)
    return _SC_SCATTER_CACHE[0](xtab, srcs, dsts)


_B2 = 1000


def _ln(v, s, b):
    mu = jnp.mean(v, axis=-1, keepdims=True)
    var = jnp.mean((v - mu) ** 2, axis=-1, keepdims=True)
    return (v - mu) / jnp.sqrt(var + 1e-5) * s + b


def _bf(a):
    return a.astype(jnp.bfloat16)


def _bdot(a, b):
    return jnp.dot(_bf(a), _bf(b), preferred_element_type=jnp.float32)


def _post_body(x_ref, agg_ref, wl_ref, w0_ref, w1_ref, b0_ref,
               outw_ref, outb_ref,
               wq_ref, bq_ref, lns_ref, lnb_ref,
               ff1_ref, ff1b_ref, ff2_ref, ff2b_ref,
               h1_ref, h1b_ref, h2_ref, h2b_ref, o_ref):
    x = x_ref[...]
    zs = []
    for m in range(NMP):
        aggm = jnp.concatenate([agg_ref[0, m], agg_ref[1, m]], axis=-1)
        hm = _bdot(aggm, wl_ref[m]) + _bdot(x, w0_ref[m]) + _bdot(x, w1_ref[m])
        hm = jnp.maximum(hm + b0_ref[m], 0.0)
        em = _bdot(hm, outw_ref[m])
        zs.append((em + outb_ref[m]) * (1.0 / NMP))

    gi = lax.broadcasted_iota(jnp.int32, (OUT, NHEAD), 0) // DH
    gh = lax.broadcasted_iota(jnp.int32, (OUT, NHEAD), 1)
    G = (gi == gh).astype(jnp.float32)
    sqrt_dh = jnp.float32(DH ** 0.5)

    for l in range(2):
        zall = jnp.concatenate(zs, axis=0)
        qkv = [_bdot(zall, wq_ref[l, t]) + bq_ref[l, t] for t in range(3)]
        qb, kb, vb = [
            [_bf(a[i * _B2:(i + 1) * _B2]).astype(jnp.float32)
             for i in range(NMP)]
            for a in qkv
        ]
        o_all = []
        for i in range(NMP):
            s_ij = [jnp.dot(qb[i] * kb[j], G, preferred_element_type=jnp.float32) / sqrt_dh
                    for j in range(NMP)]
            smax = jnp.maximum(jnp.maximum(s_ij[0], s_ij[1]), s_ij[2])
            e_ij = [jnp.exp(s - smax) for s in s_ij]
            den = e_ij[0] + e_ij[1] + e_ij[2]
            o_acc = jnp.zeros_like(zs[i])
            for j in range(NMP):
                w = jnp.dot(_bf(e_ij[j] / den).astype(jnp.float32), G.T,
                            preferred_element_type=jnp.float32)
                o_acc = o_acc + w * vb[j]
            o_all.append(o_acc)
        attn = _bdot(jnp.concatenate(o_all, axis=0), wq_ref[l, 3]) + bq_ref[l, 3]
        new_zs = [_ln(zs[i] + attn[i * _B2:(i + 1) * _B2],
                      lns_ref[l, 0], lnb_ref[l, 0])
                  for i in range(NMP)]
        zcat = jnp.concatenate(new_zs, axis=0)
        f = jnp.maximum(_bdot(zcat, ff1_ref[l]) + ff1b_ref[l], 0.0)
        f = _bdot(f, ff2_ref[l]) + ff2b_ref[l]
        zs = [_ln(new_zs[i] + f[i * _B2:(i + 1) * _B2], lns_ref[l, 1], lnb_ref[l, 1])
              for i in range(NMP)]

    pooled = (zs[0] + zs[1] + zs[2]) / jnp.float32(NMP)
    h = jnp.maximum(_bdot(pooled, h1_ref[...]) + h1b_ref[...], 0.0)
    o_ref[...] = _bdot(h, h2_ref[...]) + h2b_ref[...]


def _full(shape):
    nd = len(shape)
    return pl.BlockSpec(shape, lambda i, _n=nd: (0,) * _n)


def _post(x, agg, weights):
    in_specs = [
        pl.BlockSpec((_B2, D_IN), lambda i: (i, 0)),
        pl.BlockSpec((2, NMP, _B2, HID), lambda i: (0, 0, i, 0)),
    ] + [_full(w.shape) for w in weights]
    return pl.pallas_call(
        _post_body,
        grid=(N // _B2,),
        in_specs=in_specs,
        out_specs=pl.BlockSpec((_B2, 1), lambda i: (i, 0)),
        out_shape=jax.ShapeDtypeStruct((N, 1), jnp.float32),
    )(x, agg, *weights)


def _stack_weights(params):
    mps = [params['mp%d' % m] for m in range(NMP)]
    wl = jnp.stack([p['wl_W'] for p in mps])
    w0 = jnp.stack([p['w0_W'] for p in mps])
    w1 = jnp.stack([p['w1_W'] for p in mps])
    b0 = jnp.stack([(p['w0_b'] + p['w1_b'] + p['wl_b'])[None, :] for p in mps])
    outw = jnp.stack([p['out_W'] for p in mps])
    outb = jnp.stack([p['out_b'][None, :] for p in mps])
    encs = [params['enc%d' % l] for l in range(2)]
    wq = jnp.stack([jnp.stack([p['W' + nm] for nm in ('q', 'k', 'v', 'o')]) for p in encs])
    bq = jnp.stack([jnp.stack([p['b' + nm][None, :] for nm in ('q', 'k', 'v', 'o')]) for p in encs])
    lns = jnp.stack([jnp.stack([p['ln1_s'][None, :], p['ln2_s'][None, :]]) for p in encs])
    lnb = jnp.stack([jnp.stack([p['ln1_b'][None, :], p['ln2_b'][None, :]]) for p in encs])
    ff1 = jnp.stack([p['ff1_W'] for p in encs])
    ff1b = jnp.stack([p['ff1_b'][None, :] for p in encs])
    ff2 = jnp.stack([p['ff2_W'] for p in encs])
    ff2b = jnp.stack([p['ff2_b'][None, :] for p in encs])
    h1 = params['head1_W']
    h1b = params['head1_b'][None, :]
    h2 = params['head2_W']
    h2b = params['head2_b'][None, :]
    return (wl, w0, w1, b0, outw, outb, wq, bq, lns, lnb,
            ff1, ff1b, ff2, ff2b, h1, h1b, h2, h2b)


def _edge_lists(e0, e1, e2):
    es = [e0, e1, e2]
    pad = EDGES_PADDED - es[0].shape[1]
    base = jnp.concatenate(
        [jnp.concatenate([es[m][1].astype(jnp.int32),
                          jnp.zeros((pad,), jnp.int32)])
         for m in range(NMP)])
    srcs = jnp.stack([2 * base, 2 * base + 1])
    sink = N + jnp.arange(pad, dtype=jnp.int32) % (NPAD - N)
    dsts = jnp.concatenate(
        [jnp.concatenate([es[m][0].astype(jnp.int32), sink])
         for m in range(NMP)])
    return srcs.reshape(2, -1, CHUNK), dsts.reshape(-1, CHUNK)


def kernel(x, edge_index_rel0, edge_index_rel1, edge_index_rel2, params):
    xtab = x.reshape(2 * N, HID)
    srcs, dsts = _edge_lists(edge_index_rel0, edge_index_rel1, edge_index_rel2)
    agg = _sc_scatter(xtab, srcs, dsts)
    out = _post(x, agg, _stack_weights(params))
    return out[:, 0]
